# Initial kernel scaffold; baseline (speedup 1.0000x reference)
import jax
import jax.numpy as jnp
from jax.experimental import pallas as pl


def kernel(x, W):
    raise NotImplementedError("write your pallas kernel here")


# speedup vs baseline: 1.2848x; 1.2848x over previous
import functools

import jax
import jax.numpy as jnp
from jax.experimental import pallas as pl
from jax.experimental.pallas import tpu as pltpu

TOKENS = 16384
D_MODEL = 2048
NUM_EXPERTS = 64
TOP_K = 2
BT = 512


def _router_body(x_ref, w_ref, scores_ref, logits_ref, ew_ref, ei_ref):
    x = x_ref[...]
    w = w_ref[...]
    logits = jax.lax.dot_general(
        x, w, (((1,), (1,)), ((), ())), preferred_element_type=jnp.float32
    )
    m1 = jnp.max(logits, axis=-1, keepdims=True)
    e = jnp.exp(logits - m1)
    s = jnp.sum(e, axis=-1, keepdims=True)
    logits_ref[...] = logits
    scores_ref[...] = e / s

    iota = jax.lax.broadcasted_iota(jnp.int32, logits.shape, 1)
    i1 = jnp.min(jnp.where(logits == m1, iota, NUM_EXPERTS), axis=-1, keepdims=True)
    masked = jnp.where(iota == i1, -jnp.inf, logits)
    m2 = jnp.max(masked, axis=-1, keepdims=True)
    i2 = jnp.min(jnp.where(masked == m2, iota, NUM_EXPERTS), axis=-1, keepdims=True)
    ew_ref[:, 0:1] = 1.0 / s
    ew_ref[:, 1:2] = jnp.exp(m2 - m1) / s
    ei_ref[:, 0:1] = i1
    ei_ref[:, 1:2] = i2


@jax.jit
def kernel(x, W):
    grid = (TOKENS // BT,)
    out_shapes = (
        jax.ShapeDtypeStruct((TOKENS, NUM_EXPERTS), jnp.float32),
        jax.ShapeDtypeStruct((TOKENS, NUM_EXPERTS), jnp.float32),
        jax.ShapeDtypeStruct((TOKENS, TOP_K), jnp.float32),
        jax.ShapeDtypeStruct((TOKENS, TOP_K), jnp.int32),
    )
    scores, logits, ew, ei = pl.pallas_call(
        _router_body,
        grid=grid,
        in_specs=[
            pl.BlockSpec((BT, D_MODEL), lambda i: (i, 0)),
            pl.BlockSpec((NUM_EXPERTS, D_MODEL), lambda i: (0, 0)),
        ],
        out_specs=[
            pl.BlockSpec((BT, NUM_EXPERTS), lambda i: (i, 0)),
            pl.BlockSpec((BT, NUM_EXPERTS), lambda i: (i, 0)),
            pl.BlockSpec((BT, TOP_K), lambda i: (i, 0)),
            pl.BlockSpec((BT, TOP_K), lambda i: (i, 0)),
        ],
        out_shape=out_shapes,
        compiler_params=pltpu.CompilerParams(
            dimension_semantics=("arbitrary",),
        ),
    )(x, W)
    return scores, logits, ew, ei


# speedup vs baseline: 1.4354x; 1.1172x over previous
import functools

import jax
import jax.numpy as jnp
from jax.experimental import pallas as pl
from jax.experimental.pallas import tpu as pltpu

TOKENS = 16384
D_MODEL = 2048
NUM_EXPERTS = 64
TOP_K = 2
BT = 1024


def _router_body(x_ref, w_ref, scores_ref, logits_ref, ew_ref, ei_ref):
    x = x_ref[...]
    w = w_ref[...]
    logits = jax.lax.dot_general(
        x, w, (((1,), (1,)), ((), ())), preferred_element_type=jnp.float32
    )
    m1 = jnp.max(logits, axis=-1, keepdims=True)
    e = jnp.exp(logits - m1)
    s = jnp.sum(e, axis=-1, keepdims=True)
    logits_ref[...] = logits
    scores_ref[...] = e / s

    iota = jax.lax.broadcasted_iota(jnp.int32, logits.shape, 1)
    i1 = jnp.min(jnp.where(logits == m1, iota, NUM_EXPERTS), axis=-1, keepdims=True)
    masked = jnp.where(iota == i1, -jnp.inf, logits)
    m2 = jnp.max(masked, axis=-1, keepdims=True)
    i2 = jnp.min(jnp.where(masked == m2, iota, NUM_EXPERTS), axis=-1, keepdims=True)
    ew_ref[:, 0:1] = 1.0 / s
    ew_ref[:, 1:2] = jnp.exp(m2 - m1) / s
    ei_ref[:, 0:1] = i1
    ei_ref[:, 1:2] = i2


@jax.jit
def kernel(x, W):
    grid = (TOKENS // BT,)
    out_shapes = (
        jax.ShapeDtypeStruct((TOKENS, NUM_EXPERTS), jnp.float32),
        jax.ShapeDtypeStruct((TOKENS, NUM_EXPERTS), jnp.float32),
        jax.ShapeDtypeStruct((TOKENS, TOP_K), jnp.float32),
        jax.ShapeDtypeStruct((TOKENS, TOP_K), jnp.int32),
    )
    scores, logits, ew, ei = pl.pallas_call(
        _router_body,
        grid=grid,
        in_specs=[
            pl.BlockSpec((BT, D_MODEL), lambda i: (i, 0)),
            pl.BlockSpec((NUM_EXPERTS, D_MODEL), lambda i: (0, 0)),
        ],
        out_specs=[
            pl.BlockSpec((BT, NUM_EXPERTS), lambda i: (i, 0)),
            pl.BlockSpec((BT, NUM_EXPERTS), lambda i: (i, 0)),
            pl.BlockSpec((BT, TOP_K), lambda i: (i, 0)),
            pl.BlockSpec((BT, TOP_K), lambda i: (i, 0)),
        ],
        out_shape=out_shapes,
        compiler_params=pltpu.CompilerParams(
            dimension_semantics=("arbitrary",),
        ),
    )(x, W)
    return scores, logits, ew, ei


# speedup vs baseline: 1.4627x; 1.0191x over previous
import functools

import jax
import jax.numpy as jnp
from jax.experimental import pallas as pl
from jax.experimental.pallas import tpu as pltpu

TOKENS = 16384
D_MODEL = 2048
NUM_EXPERTS = 64
TOP_K = 2
BT = 2048


def _router_body(x_ref, w_ref, scores_ref, logits_ref, ew_ref, ei_ref):
    x = x_ref[...]
    w = w_ref[...]
    logits = jax.lax.dot_general(
        x, w, (((1,), (1,)), ((), ())), preferred_element_type=jnp.float32
    )
    m1 = jnp.max(logits, axis=-1, keepdims=True)
    e = jnp.exp(logits - m1)
    s = jnp.sum(e, axis=-1, keepdims=True)
    logits_ref[...] = logits
    scores_ref[...] = e / s

    iota = jax.lax.broadcasted_iota(jnp.int32, logits.shape, 1)
    i1 = jnp.min(jnp.where(logits == m1, iota, NUM_EXPERTS), axis=-1, keepdims=True)
    masked = jnp.where(iota == i1, -jnp.inf, logits)
    m2 = jnp.max(masked, axis=-1, keepdims=True)
    i2 = jnp.min(jnp.where(masked == m2, iota, NUM_EXPERTS), axis=-1, keepdims=True)
    ew_ref[:, 0:1] = 1.0 / s
    ew_ref[:, 1:2] = jnp.exp(m2 - m1) / s
    ei_ref[:, 0:1] = i1
    ei_ref[:, 1:2] = i2


@jax.jit
def kernel(x, W):
    grid = (TOKENS // BT,)
    out_shapes = (
        jax.ShapeDtypeStruct((TOKENS, NUM_EXPERTS), jnp.float32),
        jax.ShapeDtypeStruct((TOKENS, NUM_EXPERTS), jnp.float32),
        jax.ShapeDtypeStruct((TOKENS, TOP_K), jnp.float32),
        jax.ShapeDtypeStruct((TOKENS, TOP_K), jnp.int32),
    )
    scores, logits, ew, ei = pl.pallas_call(
        _router_body,
        grid=grid,
        in_specs=[
            pl.BlockSpec((BT, D_MODEL), lambda i: (i, 0)),
            pl.BlockSpec((NUM_EXPERTS, D_MODEL), lambda i: (0, 0)),
        ],
        out_specs=[
            pl.BlockSpec((BT, NUM_EXPERTS), lambda i: (i, 0)),
            pl.BlockSpec((BT, NUM_EXPERTS), lambda i: (i, 0)),
            pl.BlockSpec((BT, TOP_K), lambda i: (i, 0)),
            pl.BlockSpec((BT, TOP_K), lambda i: (i, 0)),
        ],
        out_shape=out_shapes,
        compiler_params=pltpu.CompilerParams(
            dimension_semantics=("arbitrary",),
        ),
    )(x, W)
    return scores, logits, ew, ei
